# baseline (speedup 1.0000x reference)
import functools

import jax
import jax.numpy as jnp
from jax import lax
from jax.experimental import pallas as pl
from jax.experimental.pallas import tpu as pltpu
from jax.experimental.pallas import tpu_sc as plsc

N = 10000
E = 320000
D = 128

NC = 2
NS = 16
NT = NC * NS
L = 16

C = 72
E_PER_TILE = E // NT
NCHUNK = E_PER_TILE // C
REM = E_PER_TILE - NCHUNK * C
NX = 2
NA = 3
NSR = 3
ND = 4
UNROLL = 12
NMAIN = ((NCHUNK - 1) // UNROLL) * UNROLL

ACH = C
NACH = N // ACH
AREM = N - NACH * ACH
ACH_ROUNDS = (NACH + NS - 1) // NS

PC = (0.98745704, -0.4084233, 0.11464988)


def _softplus16(z):
    t = jnp.exp(jnp.minimum(z, -z))
    p = jnp.float32(PC[2])
    p = p * t + jnp.float32(PC[1])
    p = p * t + jnp.float32(PC[0])
    return jnp.maximum(z, jnp.float32(0.0)) + p * t


def _sc_body(x_hbm, src_hbm, dst_hbm, attr_hbm, out_hbm,
             acc, sv0, sv1, sv2, dv0, dv1, dv2, dv3, src_r, dst_r,
             xr0, xr1, at0, at1, at2,
             sx0, sx1, sa0, sa1, sa2, ss0, ss1, ss2,
             si0, si1, si2, sj0, sj1, sj2, sj3):
    cid = lax.axis_index("c")
    sid = lax.axis_index("s")
    tid = cid * NS + sid

    src_v = (sv0, sv1, sv2)
    dst_v = (dv0, dv1, dv2, dv3)
    xr = (xr0, xr1)
    at = (at0, at1, at2)
    semx = (sx0, sx1)
    sema = (sa0, sa1, sa2)
    sems = (ss0, ss1, ss2)
    semi = (si0, si1, si2)
    semj = (sj0, sj1, sj2, sj3)

    @pl.loop(0, ACH)
    def _(r):
        for j in range(D // L):
            at0[r, pl.ds(j * L, L)] = jnp.zeros((L,), jnp.float32)

    @pl.loop(0, ACH_ROUNDS)
    def _(k):
        cidx = sid + k * NS

        @pl.when(cidx < NACH)
        def _():
            pltpu.async_copy(at0, acc.at[pl.ds(cidx * ACH, ACH)], sa0)

    @pl.when(sid == NS - 1)
    def _():
        pltpu.async_copy(at0.at[pl.ds(0, AREM)], acc.at[pl.ds(NACH * ACH, AREM)], sa1)

    base0 = tid * E_PER_TILE

    def issue_idx(c, si, sd):
        pltpu.async_copy(src_hbm.at[pl.ds(base0 + c * C, C)], src_v[si], semi[si])
        pltpu.async_copy(dst_hbm.at[pl.ds(base0 + c * C, C)], dst_v[sd], semj[sd])

    def drain_isrc(si):
        pltpu.make_async_copy(src_hbm.at[pl.ds(0, C)], src_v[si], semi[si]).wait()

    def drain_idst(sd):
        pltpu.make_async_copy(dst_hbm.at[pl.ds(0, C)], dst_v[sd], semj[sd]).wait()

    def issue_data(c, sx, sa, si):
        pltpu.async_copy(x_hbm.at[src_v[si]], xr[sx], semx[sx])
        pltpu.async_copy(attr_hbm.at[pl.ds(base0 + c * C, C)], at[sa], sema[sa])

    def drain_data(sx, sa):
        pltpu.make_async_copy(x_hbm.at[pl.ds(0, C)], xr[sx], semx[sx]).wait()
        pltpu.make_async_copy(attr_hbm.at[pl.ds(0, C)], at[sa], sema[sa]).wait()

    def drain_scatter(sa):
        pltpu.make_async_copy(attr_hbm.at[pl.ds(0, C)], at[sa], sems[sa]).wait()

    def _compute(rows, xbuf, mbuf):
        @pl.loop(0, rows, step=4)
        def _(r):
            for rr in (r, r + 1, r + 2, r + 3):
                for j in range(D // L):
                    sl = pl.ds(j * L, L)
                    z = xbuf[rr, sl] + mbuf[rr, sl]
                    mbuf[rr, sl] = _softplus16(z)

    def step(c, b, traced):
        sx, sa, sd = b % NX, b % NA, b % ND
        nsx, nsa, nsi = (b + 1) % NX, (b + 1) % NA, (b + 1) % NSR
        has1 = True if traced else (c + 1 < NCHUNK)
        has2 = True if traced else (c + 2 < NCHUNK)
        if has1:
            if traced and b < 2:

                @pl.when(c >= 2)
                def _():
                    drain_scatter(nsa)
            else:
                drain_scatter(nsa)
            drain_isrc(nsi)
            issue_data(c + 1, nsx, nsa, nsi)
        if has2:
            issue_idx(c + 2, (b + 2) % NSR, (b + 2) % ND)
        drain_data(sx, sa)
        drain_idst(sd)
        _compute(C, xr[sx], at[sa])
        pltpu.async_copy(at[sa], acc.at[dst_v[sd]], sems[sa], add=True)

    issue_idx(0, 0, 0)
    issue_idx(1, 1, 1)

    @pl.loop(0, ACH_ROUNDS)
    def _(k):
        cidx = sid + k * NS

        @pl.when(cidx < NACH)
        def _():
            pltpu.make_async_copy(attr_hbm.at[pl.ds(0, ACH)], at0, sa0).wait()

    @pl.when(sid == NS - 1)
    def _():
        pltpu.make_async_copy(
            attr_hbm.at[pl.ds(0, AREM)], at0.at[pl.ds(0, AREM)], sa1
        ).wait()

    drain_isrc(0)
    issue_data(0, 0, 0, 0)
    plsc.subcore_barrier()

    @pl.loop(0, NMAIN, step=UNROLL)
    def _(i):
        for b in range(UNROLL):
            step(i + b, b, traced=True)

    for c in range(NMAIN, NCHUNK):
        step(c, c % UNROLL, traced=False)

    for c in range(NCHUNK - NA, NCHUNK):
        drain_scatter(c % NA)

    pltpu.sync_copy(src_hbm.at[pl.ds(base0 + NCHUNK * C, REM)], src_r)
    pltpu.sync_copy(dst_hbm.at[pl.ds(base0 + NCHUNK * C, REM)], dst_r)
    pltpu.sync_copy(x_hbm.at[src_r], xr0.at[pl.ds(0, REM)])
    pltpu.sync_copy(attr_hbm.at[pl.ds(base0 + NCHUNK * C, REM)], at0.at[pl.ds(0, REM)])
    _compute(REM, xr0, at0)
    pltpu.sync_copy(at0.at[pl.ds(0, REM)], acc.at[dst_r], add=True)

    plsc.subcore_barrier()

    bufs = (at0, at1)
    semr = (sx0, sx1)
    semw = (sa0, sa1)

    def p2_read(k, b):
        pltpu.async_copy(acc.at[pl.ds((sid + k * NS) * ACH, ACH)], bufs[b], semr[b])

    def p2_write(k, b):
        pltpu.async_copy(bufs[b], out_hbm.at[cid, pl.ds((sid + k * NS) * ACH, ACH)],
                         semw[b])

    def p2_drain(sem, buf):
        pltpu.make_async_copy(attr_hbm.at[pl.ds(0, ACH)], buf, sem).wait()

    last_live = sid + (ACH_ROUNDS - 1) * NS < NACH
    p2_read(0, 0)
    for k in range(ACH_ROUNDS):
        b = k % 2
        nb = 1 - b
        if k + 1 < ACH_ROUNDS:
            if k >= 1:
                p2_drain(semw[nb], bufs[nb])
            if k + 1 == ACH_ROUNDS - 1:

                @pl.when(last_live)
                def _():
                    p2_read(k + 1, nb)
            else:
                p2_read(k + 1, nb)
        if k == ACH_ROUNDS - 1:

            @pl.when(last_live)
            def _():
                p2_drain(semr[b], bufs[b])
                p2_write(k, b)
        else:
            p2_drain(semr[b], bufs[b])
            p2_write(k, b)
    p2_drain(semw[(ACH_ROUNDS - 2) % 2], bufs[(ACH_ROUNDS - 2) % 2])

    @pl.when(last_live)
    def _():
        p2_drain(semw[(ACH_ROUNDS - 1) % 2], bufs[(ACH_ROUNDS - 1) % 2])

    @pl.when(sid == NS - 1)
    def _():
        r = NACH * ACH
        pltpu.sync_copy(acc.at[pl.ds(r, AREM)], at0.at[pl.ds(0, AREM)])
        pltpu.sync_copy(at0.at[pl.ds(0, AREM)], out_hbm.at[cid, pl.ds(r, AREM)])


def _sc_scatter(x, src, dst, attr):
    mesh = plsc.VectorSubcoreMesh(core_axis_name="c", subcore_axis_name="s")
    f = functools.partial(
        pl.kernel,
        out_type=jax.ShapeDtypeStruct((NC, N, D), jnp.float32),
        mesh=mesh,
        scratch_types=[
            pltpu.VMEM_SHARED((N, D), jnp.float32),
            pltpu.VMEM((C,), jnp.int32),
            pltpu.VMEM((C,), jnp.int32),
            pltpu.VMEM((C,), jnp.int32),
            pltpu.VMEM((C,), jnp.int32),
            pltpu.VMEM((C,), jnp.int32),
            pltpu.VMEM((C,), jnp.int32),
            pltpu.VMEM((C,), jnp.int32),
            pltpu.VMEM((REM,), jnp.int32),
            pltpu.VMEM((REM,), jnp.int32),
            pltpu.VMEM((C, D), jnp.float32),
            pltpu.VMEM((C, D), jnp.float32),
            pltpu.VMEM((C, D), jnp.float32),
            pltpu.VMEM((C, D), jnp.float32),
            pltpu.VMEM((C, D), jnp.float32),
            pltpu.SemaphoreType.DMA,
            pltpu.SemaphoreType.DMA,
            pltpu.SemaphoreType.DMA,
            pltpu.SemaphoreType.DMA,
            pltpu.SemaphoreType.DMA,
            pltpu.SemaphoreType.DMA,
            pltpu.SemaphoreType.DMA,
            pltpu.SemaphoreType.DMA,
            pltpu.SemaphoreType.DMA,
            pltpu.SemaphoreType.DMA,
            pltpu.SemaphoreType.DMA,
            pltpu.SemaphoreType.DMA,
            pltpu.SemaphoreType.DMA,
            pltpu.SemaphoreType.DMA,
            pltpu.SemaphoreType.DMA,
        ],
    )(_sc_body)
    return f(x, src, dst, attr)


def _combine_body(p0, p1, x, o):
    o[...] = p0[...] + p1[...] + x[...]


def _combine(p0, p1, x):
    blk = 1000
    return pl.pallas_call(
        _combine_body,
        out_shape=jax.ShapeDtypeStruct((N, D), jnp.float32),
        grid=(N // blk,),
        in_specs=[pl.BlockSpec((blk, D), lambda i: (i, 0))] * 3,
        out_specs=pl.BlockSpec((blk, D), lambda i: (i, 0)),
    )(p0, p1, x)


def kernel(x, edge_index, edge_attr):
    src = edge_index[0]
    dst = edge_index[1]
    partial = _sc_scatter(x, src, dst, edge_attr)
    return _combine(partial[0], partial[1], x)


# speedup vs baseline: 1.0545x; 1.0545x over previous
import functools

import jax
import jax.numpy as jnp
from jax import lax
from jax.experimental import pallas as pl
from jax.experimental.pallas import tpu as pltpu
from jax.experimental.pallas import tpu_sc as plsc

N = 10000
E = 320000
D = 128

NC = 2
NS = 16
NT = NC * NS
L = 16

C = 72
E_PER_TILE = E // NT
NCHUNK = E_PER_TILE // C
REM = E_PER_TILE - NCHUNK * C
NX = 2
NA = 3
NSR = 3
ND = 4
UNROLL = 12
NMAIN = ((NCHUNK - 1) // UNROLL) * UNROLL

ACH = C
NACH = N // ACH
AREM = N - NACH * ACH
ACH_ROUNDS = (NACH + NS - 1) // NS

PC = (0.98745704, -0.4084233, 0.11464988)


def _softplus16(z):
    t = jnp.exp(jnp.minimum(z, -z))
    p = jnp.float32(PC[2])
    p = p * t + jnp.float32(PC[1])
    p = p * t + jnp.float32(PC[0])
    return jnp.maximum(z, jnp.float32(0.0)) + p * t


def _sc_body(x_hbm, src_hbm, dst_hbm, attr_hbm, out_hbm,
             acc, sv0, sv1, sv2, dv0, dv1, dv2, dv3, src_r, dst_r,
             xr0, xr1, at0, at1, at2,
             sx0, sx1, sa0, sa1, sa2, ss0, ss1, ss2,
             si0, si1, si2, sj0, sj1, sj2, sj3):
    cid = lax.axis_index("c")
    sid = lax.axis_index("s")
    tid = cid * NS + sid

    src_v = (sv0, sv1, sv2)
    dst_v = (dv0, dv1, dv2, dv3)
    xr = (xr0, xr1)
    at = (at0, at1, at2)
    semx = (sx0, sx1)
    sema = (sa0, sa1, sa2)
    sems = (ss0, ss1, ss2)
    semi = (si0, si1, si2)
    semj = (sj0, sj1, sj2, sj3)

    @pl.loop(0, ACH)
    def _(r):
        for j in range(D // L):
            at0[r, pl.ds(j * L, L)] = jnp.zeros((L,), jnp.float32)

    @pl.loop(0, ACH_ROUNDS)
    def _(k):
        cidx = sid + k * NS

        @pl.when(cidx < NACH)
        def _():
            pltpu.async_copy(at0, acc.at[pl.ds(cidx * ACH, ACH)], sa0)

    @pl.when(sid == NS - 1)
    def _():
        pltpu.async_copy(at0.at[pl.ds(0, AREM)], acc.at[pl.ds(NACH * ACH, AREM)], sa1)

    base0 = tid * E_PER_TILE

    def issue_idx(c, si, sd):
        pltpu.async_copy(src_hbm.at[pl.ds(base0 + c * C, C)], src_v[si], semi[si])
        pltpu.async_copy(dst_hbm.at[pl.ds(base0 + c * C, C)], dst_v[sd], semj[sd])

    def drain_isrc(si):
        pltpu.make_async_copy(src_hbm.at[pl.ds(0, C)], src_v[si], semi[si]).wait()

    def drain_idst(sd):
        pltpu.make_async_copy(dst_hbm.at[pl.ds(0, C)], dst_v[sd], semj[sd]).wait()

    def issue_data(c, sx, sa, si):
        pltpu.async_copy(x_hbm.at[src_v[si]], xr[sx], semx[sx])
        pltpu.async_copy(attr_hbm.at[pl.ds(base0 + c * C, C)], at[sa], sema[sa])

    def drain_data(sx, sa):
        pltpu.make_async_copy(x_hbm.at[pl.ds(0, C)], xr[sx], semx[sx]).wait()
        pltpu.make_async_copy(attr_hbm.at[pl.ds(0, C)], at[sa], sema[sa]).wait()

    def drain_scatter(sa):
        pltpu.make_async_copy(attr_hbm.at[pl.ds(0, C)], at[sa], sems[sa]).wait()

    def _compute(rows, xbuf, mbuf):
        @pl.loop(0, rows, step=2)
        def _(r):
            for rr in (r, r + 1):
                for j in range(D // L):
                    sl = pl.ds(j * L, L)
                    z = xbuf[rr, sl] + mbuf[rr, sl]
                    mbuf[rr, sl] = _softplus16(z)

    def step(c, b, traced):
        sx, sa, sd = b % NX, b % NA, b % ND
        nsx, nsa, nsi = (b + 1) % NX, (b + 1) % NA, (b + 1) % NSR
        has1 = True if traced else (c + 1 < NCHUNK)
        has2 = True if traced else (c + 2 < NCHUNK)
        if has1:
            if traced and b < 2:

                @pl.when(c >= 2)
                def _():
                    drain_scatter(nsa)
            else:
                drain_scatter(nsa)
            drain_isrc(nsi)
            issue_data(c + 1, nsx, nsa, nsi)
        if has2:
            issue_idx(c + 2, (b + 2) % NSR, (b + 2) % ND)
        drain_data(sx, sa)
        drain_idst(sd)
        _compute(C, xr[sx], at[sa])
        pltpu.async_copy(at[sa], acc.at[dst_v[sd]], sems[sa], add=True)

    issue_idx(0, 0, 0)
    issue_idx(1, 1, 1)

    @pl.loop(0, ACH_ROUNDS)
    def _(k):
        cidx = sid + k * NS

        @pl.when(cidx < NACH)
        def _():
            pltpu.make_async_copy(attr_hbm.at[pl.ds(0, ACH)], at0, sa0).wait()

    @pl.when(sid == NS - 1)
    def _():
        pltpu.make_async_copy(
            attr_hbm.at[pl.ds(0, AREM)], at0.at[pl.ds(0, AREM)], sa1
        ).wait()

    drain_isrc(0)
    issue_data(0, 0, 0, 0)
    plsc.subcore_barrier()

    @pl.loop(0, NMAIN, step=UNROLL)
    def _(i):
        for b in range(UNROLL):
            step(i + b, b, traced=True)

    for c in range(NMAIN, NCHUNK):
        step(c, c % UNROLL, traced=False)

    for c in range(NCHUNK - NA, NCHUNK):
        drain_scatter(c % NA)

    pltpu.sync_copy(src_hbm.at[pl.ds(base0 + NCHUNK * C, REM)], src_r)
    pltpu.sync_copy(dst_hbm.at[pl.ds(base0 + NCHUNK * C, REM)], dst_r)
    pltpu.sync_copy(x_hbm.at[src_r], xr0.at[pl.ds(0, REM)])
    pltpu.sync_copy(attr_hbm.at[pl.ds(base0 + NCHUNK * C, REM)], at0.at[pl.ds(0, REM)])
    _compute(REM, xr0, at0)
    pltpu.sync_copy(at0.at[pl.ds(0, REM)], acc.at[dst_r], add=True)

    plsc.subcore_barrier()

    bufs = (at0, at1)
    semr = (sx0, sx1)
    semw = (sa0, sa1)

    def p2_read(k, b):
        pltpu.async_copy(acc.at[pl.ds((sid + k * NS) * ACH, ACH)], bufs[b], semr[b])

    def p2_write(k, b):
        pltpu.async_copy(bufs[b], out_hbm.at[cid, pl.ds((sid + k * NS) * ACH, ACH)],
                         semw[b])

    def p2_drain(sem, buf):
        pltpu.make_async_copy(attr_hbm.at[pl.ds(0, ACH)], buf, sem).wait()

    last_live = sid + (ACH_ROUNDS - 1) * NS < NACH
    p2_read(0, 0)
    for k in range(ACH_ROUNDS):
        b = k % 2
        nb = 1 - b
        if k + 1 < ACH_ROUNDS:
            if k >= 1:
                p2_drain(semw[nb], bufs[nb])
            if k + 1 == ACH_ROUNDS - 1:

                @pl.when(last_live)
                def _():
                    p2_read(k + 1, nb)
            else:
                p2_read(k + 1, nb)
        if k == ACH_ROUNDS - 1:

            @pl.when(last_live)
            def _():
                p2_drain(semr[b], bufs[b])
                p2_write(k, b)
        else:
            p2_drain(semr[b], bufs[b])
            p2_write(k, b)
    p2_drain(semw[(ACH_ROUNDS - 2) % 2], bufs[(ACH_ROUNDS - 2) % 2])

    @pl.when(last_live)
    def _():
        p2_drain(semw[(ACH_ROUNDS - 1) % 2], bufs[(ACH_ROUNDS - 1) % 2])

    @pl.when(sid == NS - 1)
    def _():
        r = NACH * ACH
        pltpu.sync_copy(acc.at[pl.ds(r, AREM)], at0.at[pl.ds(0, AREM)])
        pltpu.sync_copy(at0.at[pl.ds(0, AREM)], out_hbm.at[cid, pl.ds(r, AREM)])


def _sc_scatter(x, src, dst, attr):
    mesh = plsc.VectorSubcoreMesh(core_axis_name="c", subcore_axis_name="s")
    f = functools.partial(
        pl.kernel,
        out_type=jax.ShapeDtypeStruct((NC, N, D), jnp.float32),
        mesh=mesh,
        scratch_types=[
            pltpu.VMEM_SHARED((N, D), jnp.float32),
            pltpu.VMEM((C,), jnp.int32),
            pltpu.VMEM((C,), jnp.int32),
            pltpu.VMEM((C,), jnp.int32),
            pltpu.VMEM((C,), jnp.int32),
            pltpu.VMEM((C,), jnp.int32),
            pltpu.VMEM((C,), jnp.int32),
            pltpu.VMEM((C,), jnp.int32),
            pltpu.VMEM((REM,), jnp.int32),
            pltpu.VMEM((REM,), jnp.int32),
            pltpu.VMEM((C, D), jnp.float32),
            pltpu.VMEM((C, D), jnp.float32),
            pltpu.VMEM((C, D), jnp.float32),
            pltpu.VMEM((C, D), jnp.float32),
            pltpu.VMEM((C, D), jnp.float32),
            pltpu.SemaphoreType.DMA,
            pltpu.SemaphoreType.DMA,
            pltpu.SemaphoreType.DMA,
            pltpu.SemaphoreType.DMA,
            pltpu.SemaphoreType.DMA,
            pltpu.SemaphoreType.DMA,
            pltpu.SemaphoreType.DMA,
            pltpu.SemaphoreType.DMA,
            pltpu.SemaphoreType.DMA,
            pltpu.SemaphoreType.DMA,
            pltpu.SemaphoreType.DMA,
            pltpu.SemaphoreType.DMA,
            pltpu.SemaphoreType.DMA,
            pltpu.SemaphoreType.DMA,
            pltpu.SemaphoreType.DMA,
        ],
    )(_sc_body)
    return f(x, src, dst, attr)


def _combine_body(p0, p1, x, o):
    o[...] = p0[...] + p1[...] + x[...]


def _combine(p0, p1, x):
    blk = 1000
    return pl.pallas_call(
        _combine_body,
        out_shape=jax.ShapeDtypeStruct((N, D), jnp.float32),
        grid=(N // blk,),
        in_specs=[pl.BlockSpec((blk, D), lambda i: (i, 0))] * 3,
        out_specs=pl.BlockSpec((blk, D), lambda i: (i, 0)),
    )(p0, p1, x)


def kernel(x, edge_index, edge_attr):
    src = edge_index[0]
    dst = edge_index[1]
    partial = _sc_scatter(x, src, dst, edge_attr)
    return _combine(partial[0], partial[1], x)
